# baseline (speedup 1.0000x reference)
import functools
import math

import jax
import jax.numpy as jnp
import numpy as np
from jax import lax
from jax.experimental import pallas as pl
from jax.experimental.pallas import tpu as pltpu
from jax.experimental.pallas import tpu_sc as plsc

B = 2
H = 16
S = 4096
D = 64
HID = 1024
MAXLEN = 4096
S_BLK = 256
NS = S // S_BLK
N = B * H * S
NW = 32
PER_W = N // NW
CH = 128


def _consts():
    half = np.exp(np.arange(0, D, 2).astype(np.float64) * (-math.log(10000.0) / D))
    w64 = np.repeat(half, 2)
    cwr = np.repeat(np.cos(half), 2)
    swr = np.repeat(np.sin(half), 2)
    sign = np.tile(np.array([1.0, -1.0]), D // 2)
    swr_signed = swr * sign
    return np.stack([cwr, swr_signed]).astype(np.float32)


_CONSTS = _consts()


def _pos_body(hid_ref, gw_ref, pos_ref, idx_ref, wc_ref, carry_ref):
    s_idx = pl.program_id(1)

    @pl.when(s_idx == 0)
    def _():
        carry_ref[...] = jnp.zeros_like(carry_ref)

    hid = hid_ref[0]
    gw = gw_ref[...]
    hh = hid.astype(jnp.bfloat16)
    hl = (hid - hh.astype(jnp.float32)).astype(jnp.bfloat16)
    wh = gw.astype(jnp.bfloat16)
    wl = (gw - wh.astype(jnp.float32)).astype(jnp.bfloat16)
    dn = (((1,), (1,)), ((), ()))

    def _mm(a, b):
        return lax.dot_general(a, b, dn, preferred_element_type=jnp.float32,
                               precision=lax.Precision.DEFAULT)

    logits = _mm(hh, wh) + (_mm(hh, wl) + _mm(hl, wh))
    gates = jax.nn.sigmoid(logits)

    row = lax.broadcasted_iota(jnp.int32, (S_BLK, S_BLK), 0)
    col = lax.broadcasted_iota(jnp.int32, (S_BLK, S_BLK), 1)
    tri = (row >= col).astype(jnp.bfloat16)
    gh = gates.astype(jnp.bfloat16)
    gm = (gates - gh.astype(jnp.float32)).astype(jnp.bfloat16)
    dn2 = (((1,), (0,)), ((), ()))

    def _mm2(a, b):
        return lax.dot_general(a, b, dn2, preferred_element_type=jnp.float32,
                               precision=lax.Precision.DEFAULT)

    pos = _mm2(tri, gh) + _mm2(tri, gm)
    pos = pos + carry_ref[0:1, 0:H]
    carry_ref[0:1, 0:H] = pos[S_BLK - 1:S_BLK, :]
    pos_ref[0] = pos

    post = pos.T
    pc = jnp.clip(post, 0.0, float(MAXLEN) - 1.001)
    p0 = jnp.floor(pc)
    idx_ref[0] = p0.astype(jnp.int32)
    wc_ref[0] = pc - p0


def _positions(hidden_states, gate_w):
    return pl.pallas_call(
        _pos_body,
        grid=(B, NS),
        in_specs=[
            pl.BlockSpec((1, S_BLK, HID), lambda b, s: (b, s, 0)),
            pl.BlockSpec((H, HID), lambda b, s: (0, 0)),
        ],
        out_specs=[
            pl.BlockSpec((1, S_BLK, H), lambda b, s: (b, s, 0)),
            pl.BlockSpec((1, H, S_BLK), lambda b, s: (b, 0, s)),
            pl.BlockSpec((1, H, S_BLK), lambda b, s: (b, 0, s)),
        ],
        out_shape=[
            jax.ShapeDtypeStruct((B, S, H), jnp.float32),
            jax.ShapeDtypeStruct((B, H, S), jnp.int32),
            jax.ShapeDtypeStruct((B, H, S), jnp.float32),
        ],
        scratch_shapes=[pltpu.VMEM((8, 128), jnp.float32)],
        compiler_params=pltpu.CompilerParams(
            dimension_semantics=("arbitrary", "arbitrary"),
        ),
    )(hidden_states, gate_w)


def _sc_gather(table2, idx_flat):
    mesh = plsc.VectorSubcoreMesh(core_axis_name="c", subcore_axis_name="s")

    @functools.partial(
        pl.kernel,
        mesh=mesh,
        out_type=jax.ShapeDtypeStruct((N, 2 * D), jnp.float32),
        scratch_types=[
            pltpu.VMEM((PER_W,), jnp.int32),
            pltpu.VMEM((CH, 2 * D), jnp.float32),
            pltpu.SemaphoreType.DMA,
        ],
    )
    def k(table_hbm, idx_hbm, out_hbm, idx_v, rows_v, sem):
        wid = lax.axis_index("s") * 2 + lax.axis_index("c")
        base = wid * PER_W
        pltpu.sync_copy(idx_hbm.at[pl.ds(base, PER_W)], idx_v)

        def body(i, _):
            off = i * CH
            pltpu.async_copy(table_hbm.at[idx_v.at[pl.ds(off, CH)]],
                             rows_v, sem).wait()
            pltpu.sync_copy(rows_v, out_hbm.at[pl.ds(base + off, CH)])
            return 0

        lax.fori_loop(0, PER_W // CH, body, 0)

    return k(table2, idx_flat)


def _add_body(q_ref, k_ref, e01_ref, wc_ref, qo_ref, ko_ref):
    ef = e01_ref[0][:, :, 0:D]
    ec = e01_ref[0][:, :, D:2 * D]
    wc3 = jnp.broadcast_to(wc_ref[0][:, :, None], (H, S_BLK, D))
    pe = ef + wc3 * (ec - ef)
    qo_ref[0] = q_ref[0] + pe
    ko_ref[0] = k_ref[0] + pe


def _interp_add(q, k, e01, wc):
    qk_spec = pl.BlockSpec((1, H, S_BLK, D), lambda b, s: (b, 0, s, 0))
    return pl.pallas_call(
        _add_body,
        grid=(B, NS),
        in_specs=[
            qk_spec,
            qk_spec,
            pl.BlockSpec((1, H, S_BLK, 2 * D), lambda b, s: (b, 0, s, 0)),
            pl.BlockSpec((1, H, S_BLK), lambda b, s: (b, 0, s)),
        ],
        out_specs=[qk_spec, qk_spec],
        out_shape=[
            jax.ShapeDtypeStruct((B, H, S, D), jnp.float32),
            jax.ShapeDtypeStruct((B, H, S, D), jnp.float32),
        ],
        compiler_params=pltpu.CompilerParams(
            dimension_semantics=("parallel", "arbitrary"),
        ),
    )(q, k, e01, wc)


def kernel(q, k, hidden_states, gate_w, pos_table):
    positions, idx, wc = _positions(hidden_states, gate_w)
    table2 = jnp.concatenate(
        [pos_table, jnp.roll(pos_table, -1, axis=0)], axis=1)
    e01 = _sc_gather(table2, idx.reshape(N))
    e01 = e01.reshape(B, H, S, 2 * D)
    q_pos, k_pos = _interp_add(q, k, e01, wc)
    return (q_pos, k_pos, positions)


# speedup vs baseline: 1.0181x; 1.0181x over previous
import functools
import math

import jax
import jax.numpy as jnp
import numpy as np
from jax import lax
from jax.experimental import pallas as pl
from jax.experimental.pallas import tpu as pltpu
from jax.experimental.pallas import tpu_sc as plsc

B = 2
H = 16
S = 4096
D = 64
HID = 1024
MAXLEN = 4096
S_BLK = 256
NS = S // S_BLK
N = B * H * S
NW = 32
PER_W = N // NW
CH = 128
NB = 4


def _consts():
    half = np.exp(np.arange(0, D, 2).astype(np.float64) * (-math.log(10000.0) / D))
    w64 = np.repeat(half, 2)
    cwr = np.repeat(np.cos(half), 2)
    swr = np.repeat(np.sin(half), 2)
    sign = np.tile(np.array([1.0, -1.0]), D // 2)
    swr_signed = swr * sign
    return np.stack([cwr, swr_signed]).astype(np.float32)


_CONSTS = _consts()


def _pos_body(hid_ref, gw_ref, pos_ref, idx_ref, wc_ref, carry_ref):
    s_idx = pl.program_id(1)

    @pl.when(s_idx == 0)
    def _():
        carry_ref[...] = jnp.zeros_like(carry_ref)

    hid = hid_ref[0]
    gw = gw_ref[...]
    hh = hid.astype(jnp.bfloat16)
    hl = (hid - hh.astype(jnp.float32)).astype(jnp.bfloat16)
    wh = gw.astype(jnp.bfloat16)
    wl = (gw - wh.astype(jnp.float32)).astype(jnp.bfloat16)
    dn = (((1,), (1,)), ((), ()))

    def _mm(a, b):
        return lax.dot_general(a, b, dn, preferred_element_type=jnp.float32,
                               precision=lax.Precision.DEFAULT)

    logits = _mm(hh, wh) + (_mm(hh, wl) + _mm(hl, wh))
    gates = jax.nn.sigmoid(logits)

    row = lax.broadcasted_iota(jnp.int32, (S_BLK, S_BLK), 0)
    col = lax.broadcasted_iota(jnp.int32, (S_BLK, S_BLK), 1)
    tri = (row >= col).astype(jnp.bfloat16)
    gh = gates.astype(jnp.bfloat16)
    gm = (gates - gh.astype(jnp.float32)).astype(jnp.bfloat16)
    dn2 = (((1,), (0,)), ((), ()))

    def _mm2(a, b):
        return lax.dot_general(a, b, dn2, preferred_element_type=jnp.float32,
                               precision=lax.Precision.DEFAULT)

    pos = _mm2(tri, gh) + _mm2(tri, gm)
    pos = pos + carry_ref[0:1, 0:H]
    carry_ref[0:1, 0:H] = pos[S_BLK - 1:S_BLK, :]
    pos_ref[0] = pos

    post = pos.T
    pc = jnp.clip(post, 0.0, float(MAXLEN) - 1.001)
    p0 = jnp.floor(pc)
    idx_ref[0] = p0.astype(jnp.int32)
    wc_ref[0] = pc - p0


def _positions(hidden_states, gate_w):
    return pl.pallas_call(
        _pos_body,
        grid=(B, NS),
        in_specs=[
            pl.BlockSpec((1, S_BLK, HID), lambda b, s: (b, s, 0)),
            pl.BlockSpec((H, HID), lambda b, s: (0, 0)),
        ],
        out_specs=[
            pl.BlockSpec((1, S_BLK, H), lambda b, s: (b, s, 0)),
            pl.BlockSpec((1, H, S_BLK), lambda b, s: (b, 0, s)),
            pl.BlockSpec((1, H, S_BLK), lambda b, s: (b, 0, s)),
        ],
        out_shape=[
            jax.ShapeDtypeStruct((B, S, H), jnp.float32),
            jax.ShapeDtypeStruct((B, H, S), jnp.int32),
            jax.ShapeDtypeStruct((B, H, S), jnp.float32),
        ],
        scratch_shapes=[pltpu.VMEM((8, 128), jnp.float32)],
        compiler_params=pltpu.CompilerParams(
            dimension_semantics=("arbitrary", "arbitrary"),
        ),
    )(hidden_states, gate_w)


def _sc_gather(table2, idx_flat):
    mesh = plsc.VectorSubcoreMesh(core_axis_name="c", subcore_axis_name="s")

    @functools.partial(
        pl.kernel,
        mesh=mesh,
        out_type=jax.ShapeDtypeStruct((N, 2 * D), jnp.float32),
        scratch_types=[
            pltpu.VMEM((PER_W,), jnp.int32),
            pltpu.VMEM((NB, CH, 2 * D), jnp.float32),
            pltpu.SemaphoreType.DMA,
            pltpu.SemaphoreType.DMA,
        ],
    )
    def k(table_hbm, idx_hbm, out_hbm, idx_v, rows_v, gsem, ssem):
        wid = lax.axis_index("s") * 2 + lax.axis_index("c")
        base = wid * PER_W
        pltpu.sync_copy(idx_hbm.at[pl.ds(base, PER_W)], idx_v)

        def body(j, _):
            gcps = []
            for b in range(NB):
                off = (j * NB + b) * CH
                gcps.append(pltpu.async_copy(
                    table_hbm.at[idx_v.at[pl.ds(off, CH)]],
                    rows_v.at[b], gsem))
            for cp in gcps:
                cp.wait()
            scps = []
            for b in range(NB):
                off = (j * NB + b) * CH
                scps.append(pltpu.async_copy(
                    rows_v.at[b], out_hbm.at[pl.ds(base + off, CH)], ssem))
            for cp in scps:
                cp.wait()
            return 0

        lax.fori_loop(0, PER_W // CH // NB, body, 0)

    return k(table2, idx_flat)


def _add_body(q_ref, k_ref, e01_ref, wc_ref, qo_ref, ko_ref):
    ef = e01_ref[0][:, :, 0:D]
    ec = e01_ref[0][:, :, D:2 * D]
    wc3 = jnp.broadcast_to(wc_ref[0][:, :, None], (H, S_BLK, D))
    pe = ef + wc3 * (ec - ef)
    qo_ref[0] = q_ref[0] + pe
    ko_ref[0] = k_ref[0] + pe


def _interp_add(q, k, e01, wc):
    qk_spec = pl.BlockSpec((1, H, S_BLK, D), lambda b, s: (b, 0, s, 0))
    return pl.pallas_call(
        _add_body,
        grid=(B, NS),
        in_specs=[
            qk_spec,
            qk_spec,
            pl.BlockSpec((1, H, S_BLK, 2 * D), lambda b, s: (b, 0, s, 0)),
            pl.BlockSpec((1, H, S_BLK), lambda b, s: (b, 0, s)),
        ],
        out_specs=[qk_spec, qk_spec],
        out_shape=[
            jax.ShapeDtypeStruct((B, H, S, D), jnp.float32),
            jax.ShapeDtypeStruct((B, H, S, D), jnp.float32),
        ],
        compiler_params=pltpu.CompilerParams(
            dimension_semantics=("parallel", "arbitrary"),
        ),
    )(q, k, e01, wc)


def kernel(q, k, hidden_states, gate_w, pos_table):
    positions, idx, wc = _positions(hidden_states, gate_w)
    table2 = jnp.concatenate(
        [pos_table, jnp.roll(pos_table, -1, axis=0)], axis=1)
    e01 = _sc_gather(table2, idx.reshape(N))
    e01 = e01.reshape(B, H, S, 2 * D)
    q_pos, k_pos = _interp_add(q, k, e01, wc)
    return (q_pos, k_pos, positions)


# speedup vs baseline: 1.0185x; 1.0004x over previous
import functools
import math

import jax
import jax.numpy as jnp
import numpy as np
from jax import lax
from jax.experimental import pallas as pl
from jax.experimental.pallas import tpu as pltpu
from jax.experimental.pallas import tpu_sc as plsc

B = 2
H = 16
S = 4096
D = 64
HID = 1024
MAXLEN = 4096
S_BLK = 256
NS = S // S_BLK
N = B * H * S
NW = 32
PER_W = N // NW
CH = 64
NB = 8


def _consts():
    half = np.exp(np.arange(0, D, 2).astype(np.float64) * (-math.log(10000.0) / D))
    w64 = np.repeat(half, 2)
    cwr = np.repeat(np.cos(half), 2)
    swr = np.repeat(np.sin(half), 2)
    sign = np.tile(np.array([1.0, -1.0]), D // 2)
    swr_signed = swr * sign
    return np.stack([cwr, swr_signed]).astype(np.float32)


_CONSTS = _consts()


def _pos_body(hid_ref, gw_ref, pos_ref, idx_ref, wc_ref, carry_ref):
    s_idx = pl.program_id(1)

    @pl.when(s_idx == 0)
    def _():
        carry_ref[...] = jnp.zeros_like(carry_ref)

    hid = hid_ref[0]
    gw = gw_ref[...]
    hh = hid.astype(jnp.bfloat16)
    hl = (hid - hh.astype(jnp.float32)).astype(jnp.bfloat16)
    wh = gw.astype(jnp.bfloat16)
    wl = (gw - wh.astype(jnp.float32)).astype(jnp.bfloat16)
    dn = (((1,), (1,)), ((), ()))

    def _mm(a, b):
        return lax.dot_general(a, b, dn, preferred_element_type=jnp.float32,
                               precision=lax.Precision.DEFAULT)

    logits = _mm(hh, wh) + (_mm(hh, wl) + _mm(hl, wh))
    gates = jax.nn.sigmoid(logits)

    row = lax.broadcasted_iota(jnp.int32, (S_BLK, S_BLK), 0)
    col = lax.broadcasted_iota(jnp.int32, (S_BLK, S_BLK), 1)
    tri = (row >= col).astype(jnp.bfloat16)
    gh = gates.astype(jnp.bfloat16)
    gm = (gates - gh.astype(jnp.float32)).astype(jnp.bfloat16)
    dn2 = (((1,), (0,)), ((), ()))

    def _mm2(a, b):
        return lax.dot_general(a, b, dn2, preferred_element_type=jnp.float32,
                               precision=lax.Precision.DEFAULT)

    pos = _mm2(tri, gh) + _mm2(tri, gm)
    pos = pos + carry_ref[0:1, 0:H]
    carry_ref[0:1, 0:H] = pos[S_BLK - 1:S_BLK, :]
    pos_ref[0] = pos

    post = pos.T
    pc = jnp.clip(post, 0.0, float(MAXLEN) - 1.001)
    p0 = jnp.floor(pc)
    idx_ref[0] = p0.astype(jnp.int32)
    wc_ref[0] = pc - p0


def _positions(hidden_states, gate_w):
    return pl.pallas_call(
        _pos_body,
        grid=(B, NS),
        in_specs=[
            pl.BlockSpec((1, S_BLK, HID), lambda b, s: (b, s, 0)),
            pl.BlockSpec((H, HID), lambda b, s: (0, 0)),
        ],
        out_specs=[
            pl.BlockSpec((1, S_BLK, H), lambda b, s: (b, s, 0)),
            pl.BlockSpec((1, H, S_BLK), lambda b, s: (b, 0, s)),
            pl.BlockSpec((1, H, S_BLK), lambda b, s: (b, 0, s)),
        ],
        out_shape=[
            jax.ShapeDtypeStruct((B, S, H), jnp.float32),
            jax.ShapeDtypeStruct((B, H, S), jnp.int32),
            jax.ShapeDtypeStruct((B, H, S), jnp.float32),
        ],
        scratch_shapes=[pltpu.VMEM((8, 128), jnp.float32)],
        compiler_params=pltpu.CompilerParams(
            dimension_semantics=("arbitrary", "arbitrary"),
        ),
    )(hidden_states, gate_w)


def _sc_gather(table2, idx_flat):
    mesh = plsc.VectorSubcoreMesh(core_axis_name="c", subcore_axis_name="s")

    @functools.partial(
        pl.kernel,
        mesh=mesh,
        out_type=jax.ShapeDtypeStruct((N, 2 * D), jnp.float32),
        scratch_types=[
            pltpu.VMEM((PER_W,), jnp.int32),
            pltpu.VMEM((NB, CH, 2 * D), jnp.float32),
            pltpu.SemaphoreType.DMA,
            pltpu.SemaphoreType.DMA,
        ],
    )
    def k(table_hbm, idx_hbm, out_hbm, idx_v, rows_v, gsem, ssem):
        wid = lax.axis_index("s") * 2 + lax.axis_index("c")
        base = wid * PER_W
        pltpu.sync_copy(idx_hbm.at[pl.ds(base, PER_W)], idx_v)

        def body(j, _):
            gcps = []
            for b in range(NB):
                off = (j * NB + b) * CH
                gcps.append(pltpu.async_copy(
                    table_hbm.at[idx_v.at[pl.ds(off, CH)]],
                    rows_v.at[b], gsem))
            for cp in gcps:
                cp.wait()
            scps = []
            for b in range(NB):
                off = (j * NB + b) * CH
                scps.append(pltpu.async_copy(
                    rows_v.at[b], out_hbm.at[pl.ds(base + off, CH)], ssem))
            for cp in scps:
                cp.wait()
            return 0

        lax.fori_loop(0, PER_W // CH // NB, body, 0)

    return k(table2, idx_flat)


def _add_body(q_ref, k_ref, e01_ref, wc_ref, qo_ref, ko_ref):
    ef = e01_ref[0][:, :, 0:D]
    ec = e01_ref[0][:, :, D:2 * D]
    wc3 = jnp.broadcast_to(wc_ref[0][:, :, None], (H, S_BLK, D))
    pe = ef + wc3 * (ec - ef)
    qo_ref[0] = q_ref[0] + pe
    ko_ref[0] = k_ref[0] + pe


def _interp_add(q, k, e01, wc):
    qk_spec = pl.BlockSpec((1, H, S_BLK, D), lambda b, s: (b, 0, s, 0))
    return pl.pallas_call(
        _add_body,
        grid=(B, NS),
        in_specs=[
            qk_spec,
            qk_spec,
            pl.BlockSpec((1, H, S_BLK, 2 * D), lambda b, s: (b, 0, s, 0)),
            pl.BlockSpec((1, H, S_BLK), lambda b, s: (b, 0, s)),
        ],
        out_specs=[qk_spec, qk_spec],
        out_shape=[
            jax.ShapeDtypeStruct((B, H, S, D), jnp.float32),
            jax.ShapeDtypeStruct((B, H, S, D), jnp.float32),
        ],
        compiler_params=pltpu.CompilerParams(
            dimension_semantics=("parallel", "arbitrary"),
        ),
    )(q, k, e01, wc)


def kernel(q, k, hidden_states, gate_w, pos_table):
    positions, idx, wc = _positions(hidden_states, gate_w)
    table2 = jnp.concatenate(
        [pos_table, jnp.roll(pos_table, -1, axis=0)], axis=1)
    e01 = _sc_gather(table2, idx.reshape(N))
    e01 = e01.reshape(B, H, S, 2 * D)
    q_pos, k_pos = _interp_add(q, k, e01, wc)
    return (q_pos, k_pos, positions)
